# baseline (device time: 80623 ns/iter reference)
import jax
import jax.numpy as jnp
from jax import lax
from jax.experimental import pallas as pl
from jax.experimental.pallas import tpu as pltpu

N_DEV = 4


def kernel(x, W1, W2):
    m, d = x.shape
    f = W1.shape[1]
    M = N_DEV * m

    def body(x_ref, w1_ref, w2_ref, out_ref,
             xg_ref, agc_ref, w1b_ref, w2b_ref, acc_ref, rsc_ref,
             ag_send, ag_recv, rs_send, rs_recv):
        my = lax.axis_index("i")
        left = (my - 1) % N_DEV
        right = (my + 1) % N_DEV

        barrier_sem = pltpu.get_barrier_semaphore()
        for nbr in (left, right):
            pl.semaphore_signal(
                barrier_sem, inc=1,
                device_id=(nbr,), device_id_type=pl.DeviceIdType.MESH,
            )
        pl.semaphore_wait(barrier_sem, 2)

        xb = x_ref[:, :].astype(jnp.bfloat16)
        xg_ref[pl.ds(my * m, m), :] = xb
        agc_ref[0, :, :] = xb

        for h in range(N_DEV - 1):
            rdma = pltpu.make_async_remote_copy(
                src_ref=agc_ref.at[h],
                dst_ref=agc_ref.at[h + 1],
                send_sem=ag_send.at[h],
                recv_sem=ag_recv.at[h],
                device_id=(right,),
                device_id_type=pl.DeviceIdType.MESH,
            )
            rdma.start()
            rdma.wait()
            origin = (my - h - 1) % N_DEV
            xg_ref[pl.ds(origin * m, m), :] = agc_ref[h + 1, :, :]

        w1b_ref[:, :] = w1_ref[:, :].astype(jnp.bfloat16)
        w2b_ref[:, :] = w2_ref[:, :].astype(jnp.bfloat16)

        for rb in range(N_DEV):
            hblk = jnp.dot(
                xg_ref[pl.ds(rb * m, m), :], w1b_ref[:, :],
                preferred_element_type=jnp.float32,
            )
            hblk = hblk * (1.0 / (1.0 + jnp.exp(-hblk)))
            acc_ref[pl.ds(rb * m, m), :] = jnp.dot(
                hblk.astype(jnp.bfloat16), w2b_ref[:, :],
                preferred_element_type=jnp.float32,
            )

        for s in range(N_DEV - 1):
            c_send = (my - s - 1) % N_DEV
            if s == 0:
                rsc_ref[0, :, :] = acc_ref[pl.ds(c_send * m, m), :]
            rdma = pltpu.make_async_remote_copy(
                src_ref=rsc_ref.at[s],
                dst_ref=rsc_ref.at[s + 1],
                send_sem=rs_send.at[s],
                recv_sem=rs_recv.at[s],
                device_id=(right,),
                device_id_type=pl.DeviceIdType.MESH,
            )
            rdma.start()
            rdma.wait()
            c_recv = (my - s - 2) % N_DEV
            if s < N_DEV - 2:
                rsc_ref[s + 1, :, :] = (
                    rsc_ref[s + 1, :, :] + acc_ref[pl.ds(c_recv * m, m), :]
                )
            else:
                out_ref[:, :] = (
                    rsc_ref[s + 1, :, :] + acc_ref[pl.ds(c_recv * m, m), :]
                )

    return pl.pallas_call(
        body,
        out_shape=jax.ShapeDtypeStruct((m, d), jnp.float32),
        in_specs=[pl.BlockSpec(memory_space=pltpu.VMEM)] * 3,
        out_specs=pl.BlockSpec(memory_space=pltpu.VMEM),
        scratch_shapes=[
            pltpu.VMEM((M, d), jnp.bfloat16),
            pltpu.VMEM((N_DEV, m, d), jnp.bfloat16),
            pltpu.VMEM((d, f), jnp.bfloat16),
            pltpu.VMEM((f, d), jnp.bfloat16),
            pltpu.VMEM((M, d), jnp.float32),
            pltpu.VMEM((N_DEV, m, d), jnp.float32),
            pltpu.SemaphoreType.DMA((N_DEV - 1,)),
            pltpu.SemaphoreType.DMA((N_DEV - 1,)),
            pltpu.SemaphoreType.DMA((N_DEV - 1,)),
            pltpu.SemaphoreType.DMA((N_DEV - 1,)),
        ],
        compiler_params=pltpu.CompilerParams(collective_id=0),
    )(x, W1, W2)


# device time: 33574 ns/iter; 2.4014x vs baseline; 2.4014x over previous
import jax
import jax.numpy as jnp
from jax import lax
from jax.experimental import pallas as pl
from jax.experimental.pallas import tpu as pltpu

N_DEV = 4


def kernel(x, W1, W2):
    m, d = x.shape
    f = W1.shape[1]
    M = N_DEV * m
    q = m // 2

    def body(x_ref, w1_ref, w2_ref, out_ref,
             agcR_ref, agcL_ref, w1b_ref, w2b_ref, acc_ref,
             rscR_ref, rscL_ref,
             agR_send, agR_recv, agL_send, agL_recv,
             rsR_send, rsR_recv, rsL_send, rsL_recv):
        my = lax.axis_index("i")
        left = (my - 1) % N_DEV
        right = (my + 1) % N_DEV

        def copy(src, dst, ssem, rsem, target):
            return pltpu.make_async_remote_copy(
                src_ref=src, dst_ref=dst, send_sem=ssem, recv_sem=rsem,
                device_id=(target,), device_id_type=pl.DeviceIdType.MESH,
            )

        def compute_half(origin, src_slot, is_b):
            hblk = jnp.dot(src_slot[:, :], w1b_ref[:, :],
                           preferred_element_type=jnp.float32)
            hblk = hblk * (1.0 / (1.0 + jnp.exp(-hblk)))
            row0 = origin * m + (q if is_b else 0)
            acc_ref[pl.ds(row0, q), :] = jnp.dot(
                hblk.astype(jnp.bfloat16), w2b_ref[:, :],
                preferred_element_type=jnp.float32)

        def accA(b):
            return acc_ref[pl.ds(b * m, q), :]

        def accB(b):
            return acc_ref[pl.ds(b * m + q, q), :]

        barrier_sem = pltpu.get_barrier_semaphore()
        for nbr in (left, right):
            pl.semaphore_signal(
                barrier_sem, inc=1,
                device_id=(nbr,), device_id_type=pl.DeviceIdType.MESH,
            )
        pl.semaphore_wait(barrier_sem, 2)

        agcR_ref[0, :, :] = x_ref[pl.ds(0, q), :].astype(jnp.bfloat16)
        agcL_ref[0, :, :] = x_ref[pl.ds(q, q), :].astype(jnp.bfloat16)
        agR = [None] * (N_DEV - 1)
        agL = [None] * (N_DEV - 1)
        rsR = [None] * (N_DEV - 1)
        rsL = [None] * (N_DEV - 1)
        agR[0] = copy(agcR_ref.at[0], agcR_ref.at[1],
                      agR_send.at[0], agR_recv.at[0], right)
        agL[0] = copy(agcL_ref.at[0], agcL_ref.at[1],
                      agL_send.at[0], agL_recv.at[0], left)
        agR[0].start()
        agL[0].start()

        w1b_ref[:, :] = w1_ref[:, :].astype(jnp.bfloat16)
        w2b_ref[:, :] = w2_ref[:, :].astype(jnp.bfloat16)
        compute_half(my, agcR_ref.at[0], is_b=False)
        compute_half(my, agcL_ref.at[0], is_b=True)

        for h in range(N_DEV - 1):
            agR[h].wait()
            if h < N_DEV - 2:
                agR[h + 1] = copy(agcR_ref.at[h + 1], agcR_ref.at[h + 2],
                                  agR_send.at[h + 1], agR_recv.at[h + 1],
                                  right)
                agR[h + 1].start()
            oR = (my - h - 1) % N_DEV
            compute_half(oR, agcR_ref.at[h + 1], is_b=False)
            if h == 0:
                rscR_ref[0, :, :] = accA(oR).astype(jnp.bfloat16)
            else:
                rsR[h - 1].wait_recv()
                rscR_ref[h, :, :] = (
                    rscR_ref[h, :, :].astype(jnp.float32) + accA(oR)
                ).astype(jnp.bfloat16)
            rsR[h] = copy(rscR_ref.at[h], rscR_ref.at[h + 1],
                          rsR_send.at[h], rsR_recv.at[h], right)
            rsR[h].start()

            agL[h].wait()
            if h < N_DEV - 2:
                agL[h + 1] = copy(agcL_ref.at[h + 1], agcL_ref.at[h + 2],
                                  agL_send.at[h + 1], agL_recv.at[h + 1],
                                  left)
                agL[h + 1].start()
            oL = (my + h + 1) % N_DEV
            compute_half(oL, agcL_ref.at[h + 1], is_b=True)
            if h == 0:
                rscL_ref[0, :, :] = accB(oL).astype(jnp.bfloat16)
            else:
                rsL[h - 1].wait_recv()
                rscL_ref[h, :, :] = (
                    rscL_ref[h, :, :].astype(jnp.float32) + accB(oL)
                ).astype(jnp.bfloat16)
            rsL[h] = copy(rscL_ref.at[h], rscL_ref.at[h + 1],
                          rsL_send.at[h], rsL_recv.at[h], left)
            rsL[h].start()

        rsR[N_DEV - 2].wait_recv()
        out_ref[pl.ds(0, q), :] = (
            rscR_ref[N_DEV - 1, :, :].astype(jnp.float32) + accA(my)
        )
        rsL[N_DEV - 2].wait_recv()
        out_ref[pl.ds(q, q), :] = (
            rscL_ref[N_DEV - 1, :, :].astype(jnp.float32) + accB(my)
        )

        for s in range(N_DEV - 1):
            rsR[s].wait_send()
            rsL[s].wait_send()

    return pl.pallas_call(
        body,
        out_shape=jax.ShapeDtypeStruct((m, d), jnp.float32),
        in_specs=[pl.BlockSpec(memory_space=pltpu.VMEM)] * 3,
        out_specs=pl.BlockSpec(memory_space=pltpu.VMEM),
        scratch_shapes=[
            pltpu.VMEM((N_DEV, q, d), jnp.bfloat16),
            pltpu.VMEM((N_DEV, q, d), jnp.bfloat16),
            pltpu.VMEM((d, f), jnp.bfloat16),
            pltpu.VMEM((f, d), jnp.bfloat16),
            pltpu.VMEM((M, d), jnp.float32),
            pltpu.VMEM((N_DEV, q, d), jnp.bfloat16),
            pltpu.VMEM((N_DEV, q, d), jnp.bfloat16),
            pltpu.SemaphoreType.DMA((N_DEV - 1,)),
            pltpu.SemaphoreType.DMA((N_DEV - 1,)),
            pltpu.SemaphoreType.DMA((N_DEV - 1,)),
            pltpu.SemaphoreType.DMA((N_DEV - 1,)),
            pltpu.SemaphoreType.DMA((N_DEV - 1,)),
            pltpu.SemaphoreType.DMA((N_DEV - 1,)),
            pltpu.SemaphoreType.DMA((N_DEV - 1,)),
            pltpu.SemaphoreType.DMA((N_DEV - 1,)),
        ],
        compiler_params=pltpu.CompilerParams(collective_id=0),
    )(x, W1, W2)
